# baseline (speedup 1.0000x reference)
import functools

import jax
import jax.numpy as jnp
from jax import lax
from jax.experimental import pallas as pl
from jax.experimental.pallas import tpu as pltpu
from jax.experimental.pallas import tpu_sc as plsc

NC, NS, L = 2, 16, 16
NW = NC * NS

A_ROWS, B_ROWS, D = 4096, 2048, 1024
A_N = A_ROWS * D
B_N = B_ROWS * D
A_PW = A_N // NW
B_PW = B_N // NW

STEP = 16 * 1024
A_STEPS = A_PW // STEP
B_STEPS = B_PW // STEP
UNROLL = 8


def _compute_chunk(buf):
    @pl.loop(0, STEP // L, unroll=UNROLL)
    def _(i):
        sl = pl.ds(pl.multiple_of(i * L, L), L)
        x = buf[sl]
        buf[sl] = jnp.maximum((x + 1.0) * 2.0 + 3.0, 0.0)


def _body(a_hbm, b_hbm, out_hbm, buf):
    wid = lax.axis_index("s") * NC + lax.axis_index("c")
    a_base = pl.multiple_of(wid * A_PW, STEP)
    b_base = pl.multiple_of(wid * B_PW, STEP)
    for t in range(A_STEPS + B_STEPS):
        if t < A_STEPS:
            src = a_hbm
            s_off = a_base + t * STEP
            d_off = a_base + t * STEP
        else:
            tb = t - A_STEPS
            src = b_hbm
            s_off = b_base + tb * STEP
            d_off = A_N + b_base + tb * STEP
        pltpu.sync_copy(src.at[pl.ds(s_off, STEP)], buf)
        _compute_chunk(buf)
        pltpu.sync_copy(buf, out_hbm.at[pl.ds(d_off, STEP)])


def kernel(a, b):
    af = a.reshape(-1)
    bf = b.reshape(-1)
    mesh = plsc.VectorSubcoreMesh(
        core_axis_name="c", subcore_axis_name="s", num_cores=NC, num_subcores=NS
    )
    out = pl.kernel(
        _body,
        out_type=jax.ShapeDtypeStruct((A_N + B_N,), jnp.float32),
        mesh=mesh,
        scratch_types=[pltpu.VMEM((STEP,), jnp.float32)],
    )(af, bf)
    return out.reshape(A_ROWS + B_ROWS, D)
